# baseline (speedup 1.0000x reference)
import functools

import jax
import jax.numpy as jnp
from jax import lax
from jax.experimental import pallas as pl
from jax.experimental.pallas import tpu as pltpu
from jax.experimental.pallas import tpu_sc as plsc

_OD = 256
_NW = 32
_C = 256


def _make_sc(n):
    rows_w = n // _NW
    mesh = plsc.VectorSubcoreMesh(core_axis_name="c", subcore_axis_name="s")

    @functools.partial(
        pl.kernel, mesh=mesh,
        out_type=jax.ShapeDtypeStruct((n * _OD,), jnp.float32),
        scratch_types=[
            pltpu.VMEM((3 * _C,), jnp.float32),
            pltpu.VMEM((4 * _OD,), jnp.float32),
            pltpu.VMEM((16,), jnp.float32),
            pltpu.VMEM((_C * _OD,), jnp.float32),
        ],
    )
    def sc_kernel(xt_hbm, wtb_hbm, av_hbm, out_hbm, xv, wv, av, ov):
        wid = lax.axis_index("s") * 2 + lax.axis_index("c")
        pltpu.sync_copy(wtb_hbm, wv)
        pltpu.sync_copy(av_hbm, av)
        avec = av[...]
        base0 = wid * rows_w

        def chunk_body(ci, _):
            base = base0 + ci * _C
            for k in range(3):
                pltpu.sync_copy(xt_hbm.at[pl.ds(k * n + base, _C)],
                                xv.at[pl.ds(k * _C, _C)])

            def group_body(g, _):
                xg0 = xv[pl.ds(g * 16, 16)]
                xg1 = xv[pl.ds(_C + g * 16, 16)]
                xg2 = xv[pl.ds(2 * _C + g * 16, 16)]
                xs = [(xg0[r], xg1[r], xg2[r]) for r in range(16)]

                def j_body(j, _):
                    w0 = wv[pl.ds(16 * j, 16)]
                    w1 = wv[pl.ds(_OD + 16 * j, 16)]
                    w2 = wv[pl.ds(2 * _OD + 16 * j, 16)]
                    bj = wv[pl.ds(3 * _OD + 16 * j, 16)]
                    for r in range(16):
                        x0, x1, x2 = xs[r]
                        y = x0 * w0 + x1 * w1 + x2 * w2 + bj
                        ov[pl.ds((g * 16 + r) * _OD + 16 * j, 16)] = (
                            jnp.maximum(y, avec * y))
                    return 0

                lax.fori_loop(0, _OD // 16, j_body, 0)
                return 0

            lax.fori_loop(0, _C // 16, group_body, 0)
            pltpu.sync_copy(ov, out_hbm.at[pl.ds(base * _OD, _C * _OD)])
            return 0

        lax.fori_loop(0, rows_w // _C, chunk_body, 0)

    return sc_kernel


@jax.jit
def kernel(last, W, b, prelu_a):
    n, idim = last.shape
    odim = W.shape[0]
    xt = last.T.reshape(-1)
    wtb = jnp.concatenate([W.T, b[None, :]], axis=0).reshape(-1)
    av = jnp.broadcast_to(jnp.asarray(prelu_a, jnp.float32), (16,))
    out = _make_sc(n)(xt, wtb, av)
    return out.reshape(n, odim)


# speedup vs baseline: 8.7287x; 8.7287x over previous
import functools

import jax
import jax.numpy as jnp
from jax import lax
from jax.experimental import pallas as pl

_BN = 8192


def _body(xt_ref, wt_ref, b_ref, a_ref, o_ref):
    xt = xt_ref[:, :]
    y = lax.dot_general(xt, wt_ref[:, :],
                        dimension_numbers=(((0,), (0,)), ((), ())),
                        preferred_element_type=jnp.float32) + b_ref[:, :]
    a = a_ref[0, 0]
    o_ref[:, :] = jnp.maximum(y, a * y)


@jax.jit
def kernel(last, W, b, prelu_a):
    n, idim = last.shape
    odim = W.shape[0]
    xt = last.T
    wt = W.T
    b2 = b.reshape(1, odim)
    a2 = jnp.asarray(prelu_a, jnp.float32).reshape(1, 1)
    grid = (n // _BN,)
    return pl.pallas_call(
        _body,
        grid=grid,
        in_specs=[
            pl.BlockSpec((idim, _BN), lambda i: (0, i)),
            pl.BlockSpec((idim, odim), lambda i: (0, 0)),
            pl.BlockSpec((1, odim), lambda i: (0, 0)),
            pl.BlockSpec((1, 1), lambda i: (0, 0)),
        ],
        out_specs=pl.BlockSpec((_BN, odim), lambda i: (i, 0)),
        out_shape=jax.ShapeDtypeStruct((n, odim), jnp.float32),
    )(xt, wt, b2, a2)


# speedup vs baseline: 8.8728x; 1.0165x over previous
import functools

import jax
import jax.numpy as jnp
from jax import lax
from jax.experimental import pallas as pl
from jax.experimental.pallas import tpu as pltpu

_BN = 8192


def _body(xt_ref, wt_ref, b_ref, a_ref, o_ref):
    xt = xt_ref[:, :]
    y = lax.dot_general(xt, wt_ref[:, :],
                        dimension_numbers=(((0,), (0,)), ((), ())),
                        preferred_element_type=jnp.float32) + b_ref[:, :]
    a = a_ref[0, 0]
    o_ref[:, :] = jnp.maximum(y, a * y)


@jax.jit
def kernel(last, W, b, prelu_a):
    n, idim = last.shape
    odim = W.shape[0]
    xt = last.T
    wt = W.T
    b2 = b.reshape(1, odim)
    a2 = jnp.asarray(prelu_a, jnp.float32).reshape(1, 1)
    grid = (n // _BN,)
    return pl.pallas_call(
        _body,
        grid=grid,
        in_specs=[
            pl.BlockSpec((idim, _BN), lambda i: (0, i)),
            pl.BlockSpec((idim, odim), lambda i: (0, 0)),
            pl.BlockSpec((1, odim), lambda i: (0, 0)),
            pl.BlockSpec((1, 1), lambda i: (0, 0)),
        ],
        out_specs=pl.BlockSpec((_BN, odim), lambda i: (i, 0)),
        out_shape=jax.ShapeDtypeStruct((n, odim), jnp.float32),
        compiler_params=pltpu.CompilerParams(
            allow_input_fusion=[True, False, False, False]),
    )(xt, wt, b2, a2)
